# Initial kernel scaffold; baseline (speedup 1.0000x reference)
import jax
import jax.numpy as jnp
from jax.experimental import pallas as pl


def kernel(x, my_mask2, ln_g, ln_b, Wh, bh, Wqk, bqk, gamma, beta, rel_emb, Wout, bout):
    raise NotImplementedError("write your pallas kernel here")


# speedup vs baseline: 30.4355x; 30.4355x over previous
import numpy as np
import jax
import jax.numpy as jnp
from jax import lax
from jax.experimental import pallas as pl
from jax.experimental.pallas import tpu as pltpu

B, S, DIM = 20, 500, 300
HID = 600
QK = 128
ROT = 32
NB = 32


def _bias_kernel(emb_ref, bucket_ref, o_ref):
    emb = jnp.broadcast_to(emb_ref[...], (S, NB))
    o_ref[...] = jnp.take_along_axis(emb, bucket_ref[...], axis=1)


def _gau_kernel(x_ref, mask2_ref, bias_ref, lng_ref, lnb_ref,
                Whv_ref, bhv_ref, Whg_ref, bhg_ref,
                Wqk_ref, bqk_ref, gamma_ref, beta_ref,
                cs_ref, sn_ref, R_ref,
                A4_ref, B4_ref, A4T_ref, B4T_ref,
                Wout_ref, bout_ref, o_ref):
    f32 = jnp.float32
    xb = x_ref[0]

    m = jnp.mean(xb, axis=-1, keepdims=True)
    xc = xb - m
    var = jnp.mean(xc * xc, axis=-1, keepdims=True)
    nx = xc * lax.rsqrt(var + 1e-5) * lng_ref[...] + lnb_ref[...]

    shifted = jnp.concatenate([jnp.zeros((1, DIM), f32), nx[:-1]], axis=0)
    lane = lax.broadcasted_iota(jnp.int32, (1, DIM), 1)
    nx2 = jnp.where(lane < DIM // 2, shifted, nx)

    v = jnp.dot(nx2, Whv_ref[...]) + bhv_ref[...]
    v = v * (1.0 / (1.0 + jnp.exp(-v)))
    gate = jnp.dot(nx2, Whg_ref[...]) + bhg_ref[...]
    gate = gate * (1.0 / (1.0 + jnp.exp(-gate)))

    absg = jnp.abs(gate)
    gmax = jnp.max(jnp.max(gate, axis=0, keepdims=True), axis=1, keepdims=True)
    floor_g = jnp.floor(gmax)
    lo = jnp.zeros((1, 1), f32)
    hi = jnp.full((1, 1), 128.0, f32)
    for _ in range(8):
        mid = jnp.floor((lo + hi + 1.0) * 0.5)
        ge = jnp.where(absg >= mid, 1.0, 0.0)
        cnt = jnp.sum(jnp.sum(ge, axis=0, keepdims=True), axis=1, keepdims=True)
        ok = cnt > 90000.0
        lo = jnp.where(ok, mid, lo)
        hi = jnp.where(ok, hi, mid - 1.0)
    trim = jnp.maximum(jnp.minimum(lo, floor_g), 1.0)

    ind = jnp.where(absg >= trim, 1.0, 0.0)
    counts = jnp.dot(A4_ref[...], jnp.dot(ind, B4_ref[...]))
    cmax = jnp.max(jnp.max(counts, axis=0, keepdims=True), axis=1, keepdims=True)
    t2star = jnp.zeros((1, 1), f32)
    for t in range(1, 17):
        ct = jnp.sum(jnp.sum(jnp.where(counts >= float(t), 1.0, 0.0),
                             axis=0, keepdims=True), axis=1, keepdims=True)
        t2star = t2star + jnp.where(ct > 3600.0, 1.0, 0.0)
    t2 = jnp.where(t2star >= 0.5, t2star, cmax)
    bv = jnp.where(counts >= t2, 1.0, 0.25)
    gm = jnp.dot(A4T_ref[...], jnp.dot(bv, B4T_ref[...]))

    qk = jnp.dot(nx2, Wqk_ref[...]) + bqk_ref[...]
    qk = qk * (1.0 / (1.0 + jnp.exp(-qk)))
    cs = cs_ref[...]
    sn = sn_ref[...]
    R = R_ref[...]
    q0 = qk * gamma_ref[0:1, :] + beta_ref[0:1, :]
    k0 = qk * gamma_ref[1:2, :] + beta_ref[1:2, :]
    q = q0 * cs + jnp.dot(q0, R) * sn
    k = k0 * cs + jnp.dot(k0, R) * sn
    sim = lax.dot_general(q, k, (((1,), (1,)), ((), ()))) + bias_ref[...]
    a = jnp.maximum(sim, 0.0) * (1.0 / S)
    attn = a * a * mask2_ref[...]
    out = jnp.dot(attn, v)
    out = gm * out * gate
    o_ref[0] = jnp.dot(out, Wout_ref[...]) + bout_ref[...] + xb


def kernel(x, my_mask2, ln_g, ln_b, Wh, bh, Wqk, bqk, gamma, beta, rel_emb, Wout, bout):
    f32 = jnp.float32

    inv = 1.0 / (10000.0 ** (jnp.arange(0, ROT, 2, dtype=f32) / ROT))
    fr = jnp.repeat(jnp.arange(S, dtype=f32)[:, None] * inv[None, :], 2, axis=-1)
    cs = jnp.concatenate([jnp.cos(fr), jnp.ones((S, QK - ROT), f32)], axis=1)
    sn = jnp.concatenate([jnp.sin(fr), jnp.zeros((S, QK - ROT), f32)], axis=1)
    Rnp = np.zeros((QK, QK), np.float32)
    for i in range(0, ROT, 2):
        Rnp[i + 1, i] = -1.0
        Rnp[i, i + 1] = 1.0
    R = jnp.asarray(Rnp)

    A4np = np.zeros((100, S), np.float32)
    for r in range(4):
        A4np[np.arange(100), 5 * np.arange(100) + r] = 1.0
    B4np = np.zeros((HID, 120), np.float32)
    for c in range(4):
        B4np[5 * np.arange(120) + c, np.arange(120)] = 1.0
    A4 = jnp.asarray(A4np)
    B4 = jnp.asarray(B4np)
    A4T = jnp.asarray(np.ascontiguousarray(A4np.T))
    B4T = jnp.asarray(np.ascontiguousarray(B4np.T))

    nb = NB // 2
    pos = jnp.arange(S)
    n = pos[:, None] - pos[None, :]
    ret = (n < 0).astype(jnp.int32) * nb
    na = jnp.abs(n)
    max_exact = nb // 2
    vil = max_exact + (jnp.log(jnp.maximum(na, 1).astype(f32) / max_exact)
                       / np.float32(np.log(128.0 / max_exact))
                       * (nb - max_exact)).astype(jnp.int32)
    vil = jnp.minimum(vil, nb - 1)
    bucket = (ret + jnp.where(na < max_exact, na, vil)).astype(jnp.int32)

    emb_row = (rel_emb[:, 0] * np.float32(QK ** 0.5)).reshape(1, NB)
    bias = pl.pallas_call(
        _bias_kernel,
        out_shape=jax.ShapeDtypeStruct((S, S), f32),
        name="t5_bias",
    )(emb_row, bucket)

    Whv, Whg = Wh[:, :HID], Wh[:, HID:]
    bhv, bhg = bh[:HID].reshape(1, HID), bh[HID:].reshape(1, HID)
    lng, lnb = ln_g.reshape(1, DIM), ln_b.reshape(1, DIM)
    bqk2 = bqk.reshape(1, QK)
    bout2 = bout.reshape(1, DIM)

    def full(a):
        nd = a.ndim
        return pl.BlockSpec(a.shape, lambda b, _n=nd: (0,) * _n)

    consts = (my_mask2, bias, lng, lnb, Whv, bhv, Whg, bhg, Wqk, bqk2,
              gamma, beta, cs, sn, R, A4, B4, A4T, B4T, Wout, bout2)
    in_specs = [pl.BlockSpec((1, S, DIM), lambda b: (b, 0, 0))]
    in_specs += [full(a) for a in consts]

    out = pl.pallas_call(
        _gau_kernel,
        grid=(B,),
        in_specs=in_specs,
        out_specs=pl.BlockSpec((1, S, DIM), lambda b: (b, 0, 0)),
        out_shape=jax.ShapeDtypeStruct((B, S, DIM), f32),
        compiler_params=pltpu.CompilerParams(
            dimension_semantics=("arbitrary",),
            vmem_limit_bytes=48 * 1024 * 1024,
        ),
        name="gau_fused",
    )(x, *consts)
    return out


# speedup vs baseline: 31.3574x; 1.0303x over previous
import numpy as np
import jax
import jax.numpy as jnp
from jax import lax
from jax.experimental import pallas as pl
from jax.experimental.pallas import tpu as pltpu

B, S, DIM = 20, 500, 300
HID = 600
QK = 128
ROT = 32
NB = 32


def _bias_kernel(emb_ref, bucket_ref, o_ref):
    emb = jnp.broadcast_to(emb_ref[...], (S, NB))
    o_ref[...] = jnp.take_along_axis(emb, bucket_ref[...], axis=1)


def _gau_kernel(x_ref, mask2_ref, bias_ref, lng_ref, lnb_ref,
                Whv_ref, bhv_ref, Whg_ref, bhg_ref,
                Wqk_ref, bqk_ref, gamma_ref, beta_ref,
                cs_ref, sn_ref, R_ref,
                A4_ref, B4_ref, A4T_ref, B4T_ref,
                Wout_ref, bout_ref, o_ref):
    f32 = jnp.float32
    xb = x_ref[0]

    m = jnp.mean(xb, axis=-1, keepdims=True)
    xc = xb - m
    var = jnp.mean(xc * xc, axis=-1, keepdims=True)
    nx = xc * lax.rsqrt(var + 1e-5) * lng_ref[...] + lnb_ref[...]

    shifted = jnp.concatenate([jnp.zeros((1, DIM), f32), nx[:-1]], axis=0)
    lane = lax.broadcasted_iota(jnp.int32, (1, DIM), 1)
    nx2 = jnp.where(lane < DIM // 2, shifted, nx)

    bf16 = jnp.bfloat16
    nx2b = nx2.astype(bf16)
    v = jnp.dot(nx2b, Whv_ref[...], preferred_element_type=f32) + bhv_ref[...]
    v = v * (1.0 / (1.0 + jnp.exp(-v)))
    gate = jnp.dot(nx2b, Whg_ref[...], preferred_element_type=f32) + bhg_ref[...]
    gate = gate * (1.0 / (1.0 + jnp.exp(-gate)))

    absg = jnp.abs(gate)
    gmax = jnp.max(jnp.max(gate, axis=0, keepdims=True), axis=1, keepdims=True)
    floor_g = jnp.floor(gmax)
    lo = jnp.zeros((1, 1), f32)
    hi = jnp.full((1, 1), 128.0, f32)
    for _ in range(8):
        mid = jnp.floor((lo + hi + 1.0) * 0.5)
        ge = jnp.where(absg >= mid, 1.0, 0.0)
        cnt = jnp.sum(jnp.sum(ge, axis=0, keepdims=True), axis=1, keepdims=True)
        ok = cnt > 90000.0
        lo = jnp.where(ok, mid, lo)
        hi = jnp.where(ok, hi, mid - 1.0)
    trim = jnp.maximum(jnp.minimum(lo, floor_g), 1.0)

    ind = jnp.where(absg >= trim, 1.0, 0.0).astype(jnp.bfloat16)
    counts = jnp.dot(A4_ref[...],
                     jnp.dot(ind, B4_ref[...], preferred_element_type=f32).astype(jnp.bfloat16),
                     preferred_element_type=f32)
    cmax = jnp.max(jnp.max(counts, axis=0, keepdims=True), axis=1, keepdims=True)
    t2star = jnp.zeros((1, 1), f32)
    for t in range(1, 17):
        ct = jnp.sum(jnp.sum(jnp.where(counts >= float(t), 1.0, 0.0),
                             axis=0, keepdims=True), axis=1, keepdims=True)
        t2star = t2star + jnp.where(ct > 3600.0, 1.0, 0.0)
    t2 = jnp.where(t2star >= 0.5, t2star, cmax)
    bv = jnp.where(counts >= t2, 1.0, 0.25).astype(jnp.bfloat16)
    gm = jnp.dot(A4T_ref[...],
                 jnp.dot(bv, B4T_ref[...], preferred_element_type=f32).astype(jnp.bfloat16),
                 preferred_element_type=f32)

    qk = jnp.dot(nx2b, Wqk_ref[...], preferred_element_type=f32) + bqk_ref[...]
    qk = qk * (1.0 / (1.0 + jnp.exp(-qk)))
    cs = cs_ref[...]
    sn = sn_ref[...]
    R = R_ref[...]
    q0 = qk * gamma_ref[0:1, :] + beta_ref[0:1, :]
    k0 = qk * gamma_ref[1:2, :] + beta_ref[1:2, :]
    q0b = q0.astype(bf16)
    k0b = k0.astype(bf16)
    q = (q0 * cs + jnp.dot(q0b, R, preferred_element_type=f32) * sn).astype(bf16)
    k = (k0 * cs + jnp.dot(k0b, R, preferred_element_type=f32) * sn).astype(bf16)
    sim = lax.dot_general(q, k, (((1,), (1,)), ((), ())),
                          preferred_element_type=f32) + bias_ref[...]
    a = jnp.maximum(sim, 0.0) * (1.0 / S)
    attn = (a * a * mask2_ref[...]).astype(bf16)
    out = jnp.dot(attn, v.astype(bf16), preferred_element_type=f32)
    out = (gm * out * gate).astype(bf16)
    o_ref[0] = jnp.dot(out, Wout_ref[...], preferred_element_type=f32) + bout_ref[...] + xb


def kernel(x, my_mask2, ln_g, ln_b, Wh, bh, Wqk, bqk, gamma, beta, rel_emb, Wout, bout):
    f32 = jnp.float32

    inv = 1.0 / (10000.0 ** (jnp.arange(0, ROT, 2, dtype=f32) / ROT))
    fr = jnp.repeat(jnp.arange(S, dtype=f32)[:, None] * inv[None, :], 2, axis=-1)
    cs = jnp.concatenate([jnp.cos(fr), jnp.ones((S, QK - ROT), f32)], axis=1)
    sn = jnp.concatenate([jnp.sin(fr), jnp.zeros((S, QK - ROT), f32)], axis=1)
    Rnp = np.zeros((QK, QK), np.float32)
    for i in range(0, ROT, 2):
        Rnp[i + 1, i] = -1.0
        Rnp[i, i + 1] = 1.0
    R = jnp.asarray(Rnp, jnp.bfloat16)

    A4np = np.zeros((100, S), np.float32)
    for r in range(4):
        A4np[np.arange(100), 5 * np.arange(100) + r] = 1.0
    B4np = np.zeros((HID, 120), np.float32)
    for c in range(4):
        B4np[5 * np.arange(120) + c, np.arange(120)] = 1.0
    A4 = jnp.asarray(A4np, jnp.bfloat16)
    B4 = jnp.asarray(B4np, jnp.bfloat16)
    A4T = jnp.asarray(np.ascontiguousarray(A4np.T), jnp.bfloat16)
    B4T = jnp.asarray(np.ascontiguousarray(B4np.T), jnp.bfloat16)

    nb = NB // 2
    pos = jnp.arange(S)
    n = pos[:, None] - pos[None, :]
    ret = (n < 0).astype(jnp.int32) * nb
    na = jnp.abs(n)
    max_exact = nb // 2
    vil = max_exact + (jnp.log(jnp.maximum(na, 1).astype(f32) / max_exact)
                       / np.float32(np.log(128.0 / max_exact))
                       * (nb - max_exact)).astype(jnp.int32)
    vil = jnp.minimum(vil, nb - 1)
    bucket = (ret + jnp.where(na < max_exact, na, vil)).astype(jnp.int32)

    emb_row = (rel_emb[:, 0] * np.float32(QK ** 0.5)).reshape(1, NB)
    bias = pl.pallas_call(
        _bias_kernel,
        out_shape=jax.ShapeDtypeStruct((S, S), f32),
        name="t5_bias",
    )(emb_row, bucket)

    bf16 = jnp.bfloat16
    Whv, Whg = Wh[:, :HID].astype(bf16), Wh[:, HID:].astype(bf16)
    bhv, bhg = bh[:HID].reshape(1, HID), bh[HID:].reshape(1, HID)
    lng, lnb = ln_g.reshape(1, DIM), ln_b.reshape(1, DIM)
    bqk2 = bqk.reshape(1, QK)
    bout2 = bout.reshape(1, DIM)

    def full(a):
        nd = a.ndim
        return pl.BlockSpec(a.shape, lambda b, _n=nd: (0,) * _n)

    consts = (my_mask2, bias, lng, lnb, Whv, bhv, Whg, bhg, Wqk.astype(bf16), bqk2,
              gamma, beta, cs, sn, R, A4, B4, A4T, B4T, Wout.astype(jnp.bfloat16), bout2)
    in_specs = [pl.BlockSpec((1, S, DIM), lambda b: (b, 0, 0))]
    in_specs += [full(a) for a in consts]

    out = pl.pallas_call(
        _gau_kernel,
        grid=(B,),
        in_specs=in_specs,
        out_specs=pl.BlockSpec((1, S, DIM), lambda b: (b, 0, 0)),
        out_shape=jax.ShapeDtypeStruct((B, S, DIM), f32),
        compiler_params=pltpu.CompilerParams(
            dimension_semantics=("arbitrary",),
            vmem_limit_bytes=48 * 1024 * 1024,
        ),
        name="gau_fused",
    )(x, *consts)
    return out


# speedup vs baseline: 39.0960x; 1.2468x over previous
import numpy as np
import jax
import jax.numpy as jnp
from jax import lax
from jax.experimental import pallas as pl
from jax.experimental.pallas import tpu as pltpu

B, S, DIM = 20, 500, 300
HID = 600
QK = 128
ROT = 32
NB = 32


def _bias_kernel(emb_ref, bucket_ref, o_ref):
    emb = jnp.broadcast_to(emb_ref[...], (S, NB))
    o_ref[...] = jnp.take_along_axis(emb, bucket_ref[...], axis=1)


def _gau_kernel(x_ref, mask2_ref, bias_ref, lng_ref, lnb_ref,
                Whv_ref, bhv_ref, Whg_ref, bhg_ref,
                Wqk_ref, bqk_ref, gamma_ref, beta_ref,
                cs_ref, sn_ref, R_ref,
                A4_ref, B4_ref, A4T_ref, B4T_ref,
                Wout_ref, bout_ref, o_ref):
    f32 = jnp.float32
    xb = x_ref[0]

    m = jnp.mean(xb, axis=-1, keepdims=True)
    var = jnp.mean(xb * xb, axis=-1, keepdims=True) - m * m
    nx = (xb - m) * lax.rsqrt(var + 1e-5) * lng_ref[...] + lnb_ref[...]

    shifted = jnp.concatenate([jnp.zeros((1, DIM), f32), nx[:-1]], axis=0)
    lane = lax.broadcasted_iota(jnp.int32, (1, DIM), 1)
    nx2 = jnp.where(lane < DIM // 2, shifted, nx)

    bf16 = jnp.bfloat16
    nx2b = nx2.astype(bf16)
    v = jnp.dot(nx2b, Whv_ref[...], preferred_element_type=f32) + bhv_ref[...]
    v = v * (1.0 / (1.0 + jnp.exp(-v)))
    gate = jnp.dot(nx2b, Whg_ref[...], preferred_element_type=f32) + bhg_ref[...]
    gate = (gate * (1.0 / (1.0 + jnp.exp(-gate)))).astype(bf16)

    one_b = jnp.bfloat16(1.0)
    zero_b = jnp.bfloat16(0.0)
    ones_row = jnp.ones((1, S), bf16)
    absg = jnp.abs(gate)
    gmax = jnp.max(jnp.max(gate, axis=0, keepdims=True), axis=1, keepdims=True)
    floor_g = jnp.floor(gmax).astype(f32)

    def _count_ge(thresh_bf):
        geb = jnp.where(absg >= thresh_bf, one_b, zero_b)
        c = jnp.dot(ones_row, geb, preferred_element_type=f32)
        return jnp.sum(c, axis=1, keepdims=True)

    cnt1 = _count_ge(one_b)

    def _search():
        lo = jnp.ones((1, 1), f32)
        hi = jnp.full((1, 1), 128.0, f32)
        for _ in range(7):
            mid = jnp.floor((lo + hi + 1.0) * 0.5)
            cnt = _count_ge(mid.astype(bf16))
            ok = cnt > 90000.0
            lo = jnp.where(ok, mid, lo)
            hi = jnp.where(ok, hi, mid - 1.0)
        return lo

    tstar = lax.cond(cnt1[0, 0] > 90000.0, _search,
                     lambda: jnp.zeros((1, 1), f32))
    trim = jnp.maximum(jnp.minimum(tstar, floor_g), 1.0)

    ind = jnp.where(absg >= trim.astype(bf16), one_b, zero_b)
    counts = jnp.dot(A4_ref[...],
                     jnp.dot(ind, B4_ref[...], preferred_element_type=f32).astype(jnp.bfloat16),
                     preferred_element_type=f32)
    cmax = jnp.max(jnp.max(counts, axis=0, keepdims=True), axis=1, keepdims=True)
    t2star = jnp.zeros((1, 1), f32)
    for t in range(1, 17):
        ct = jnp.sum(jnp.sum(jnp.where(counts >= float(t), 1.0, 0.0),
                             axis=0, keepdims=True), axis=1, keepdims=True)
        t2star = t2star + jnp.where(ct > 3600.0, 1.0, 0.0)
    t2 = jnp.where(t2star >= 0.5, t2star, cmax)
    bv = jnp.where(counts >= t2, 1.0, 0.25).astype(jnp.bfloat16)
    gm = jnp.dot(A4T_ref[...],
                 jnp.dot(bv, B4T_ref[...], preferred_element_type=f32).astype(jnp.bfloat16),
                 preferred_element_type=f32).astype(jnp.bfloat16)

    qk = jnp.dot(nx2b, Wqk_ref[...], preferred_element_type=f32) + bqk_ref[...]
    qk = qk * (1.0 / (1.0 + jnp.exp(-qk)))
    cs = cs_ref[...]
    sn = sn_ref[...]
    R = R_ref[...]
    q0 = qk * gamma_ref[0:1, :] + beta_ref[0:1, :]
    k0 = qk * gamma_ref[1:2, :] + beta_ref[1:2, :]
    q0b = q0.astype(bf16)
    k0b = k0.astype(bf16)
    q = (q0 * cs + jnp.dot(q0b, R, preferred_element_type=f32) * sn).astype(bf16)
    k = (k0 * cs + jnp.dot(k0b, R, preferred_element_type=f32) * sn).astype(bf16)
    sim = lax.dot_general(q, k, (((1,), (1,)), ((), ())),
                          preferred_element_type=f32) + bias_ref[...]
    a = (jnp.maximum(sim, 0.0) * (1.0 / S)).astype(bf16)
    attn = a * a * mask2_ref[...]
    out = jnp.dot(attn, v.astype(bf16), preferred_element_type=f32).astype(bf16)
    out = gm * out * gate
    o_ref[0] = jnp.dot(out, Wout_ref[...], preferred_element_type=f32) + bout_ref[...] + xb


def kernel(x, my_mask2, ln_g, ln_b, Wh, bh, Wqk, bqk, gamma, beta, rel_emb, Wout, bout):
    f32 = jnp.float32

    inv = 1.0 / (10000.0 ** (jnp.arange(0, ROT, 2, dtype=f32) / ROT))
    fr = jnp.repeat(jnp.arange(S, dtype=f32)[:, None] * inv[None, :], 2, axis=-1)
    cs = jnp.concatenate([jnp.cos(fr), jnp.ones((S, QK - ROT), f32)], axis=1)
    sn = jnp.concatenate([jnp.sin(fr), jnp.zeros((S, QK - ROT), f32)], axis=1)
    Rnp = np.zeros((QK, QK), np.float32)
    for i in range(0, ROT, 2):
        Rnp[i + 1, i] = -1.0
        Rnp[i, i + 1] = 1.0
    R = jnp.asarray(Rnp, jnp.bfloat16)

    A4np = np.zeros((100, S), np.float32)
    for r in range(4):
        A4np[np.arange(100), 5 * np.arange(100) + r] = 1.0
    B4np = np.zeros((HID, 120), np.float32)
    for c in range(4):
        B4np[5 * np.arange(120) + c, np.arange(120)] = 1.0
    A4 = jnp.asarray(A4np, jnp.bfloat16)
    B4 = jnp.asarray(B4np, jnp.bfloat16)
    A4T = jnp.asarray(np.ascontiguousarray(A4np.T), jnp.bfloat16)
    B4T = jnp.asarray(np.ascontiguousarray(B4np.T), jnp.bfloat16)

    nb = NB // 2
    pos = jnp.arange(S)
    n = pos[:, None] - pos[None, :]
    ret = (n < 0).astype(jnp.int32) * nb
    na = jnp.abs(n)
    max_exact = nb // 2
    vil = max_exact + (jnp.log(jnp.maximum(na, 1).astype(f32) / max_exact)
                       / np.float32(np.log(128.0 / max_exact))
                       * (nb - max_exact)).astype(jnp.int32)
    vil = jnp.minimum(vil, nb - 1)
    bucket = (ret + jnp.where(na < max_exact, na, vil)).astype(jnp.int32)

    emb_row = (rel_emb[:, 0] * np.float32(QK ** 0.5)).reshape(1, NB)
    bias = pl.pallas_call(
        _bias_kernel,
        out_shape=jax.ShapeDtypeStruct((S, S), f32),
        name="t5_bias",
    )(emb_row, bucket)

    bf16 = jnp.bfloat16
    Whv, Whg = Wh[:, :HID].astype(bf16), Wh[:, HID:].astype(bf16)
    bhv, bhg = bh[:HID].reshape(1, HID), bh[HID:].reshape(1, HID)
    lng, lnb = ln_g.reshape(1, DIM), ln_b.reshape(1, DIM)
    bqk2 = bqk.reshape(1, QK)
    bout2 = bout.reshape(1, DIM)

    def full(a):
        nd = a.ndim
        return pl.BlockSpec(a.shape, lambda b, _n=nd: (0,) * _n)

    consts = (my_mask2.astype(bf16), bias, lng, lnb, Whv, bhv, Whg, bhg, Wqk.astype(bf16), bqk2,
              gamma, beta, cs, sn, R, A4, B4, A4T, B4T, Wout.astype(jnp.bfloat16), bout2)
    in_specs = [pl.BlockSpec((1, S, DIM), lambda b: (b, 0, 0))]
    in_specs += [full(a) for a in consts]

    out = pl.pallas_call(
        _gau_kernel,
        grid=(B,),
        in_specs=in_specs,
        out_specs=pl.BlockSpec((1, S, DIM), lambda b: (b, 0, 0)),
        out_shape=jax.ShapeDtypeStruct((B, S, DIM), f32),
        compiler_params=pltpu.CompilerParams(
            dimension_semantics=("arbitrary",),
            vmem_limit_bytes=48 * 1024 * 1024,
        ),
        name="gau_fused",
    )(x, *consts)
    return out


# speedup vs baseline: 43.6959x; 1.1177x over previous
import numpy as np
import jax
import jax.numpy as jnp
from jax import lax
from jax.experimental import pallas as pl
from jax.experimental.pallas import tpu as pltpu

B, S, DIM = 20, 500, 300
HID = 600
QK = 128
ROT = 32
NB = 32


def _bias_kernel(emb_ref, bucket_ref, o_ref):
    emb = jnp.broadcast_to(emb_ref[...], (S, NB))
    o_ref[...] = jnp.take_along_axis(emb, bucket_ref[...], axis=1)


G = 2


def _gau_kernel(x_ref, mask2_ref, bias_ref, lng_ref, lnb_ref,
                Whv_ref, bhv_ref, Whg_ref, bhg_ref,
                Wqk_ref, bqk_ref, gamma_ref, beta_ref,
                cs_ref, sn_ref, R_ref,
                A4_ref, B4_ref, A4T_ref, B4T_ref,
                Wout_ref, bout_ref, o_ref):
    f32 = jnp.float32
    bf16 = jnp.bfloat16
    rng = range(G)
    one_b = jnp.bfloat16(1.0)
    zero_b = jnp.bfloat16(0.0)
    ones_row = jnp.ones((1, S), bf16)

    def _count_ge(absg, thresh_bf):
        geb = jnp.where(absg >= thresh_bf, one_b, zero_b)
        c = jnp.dot(ones_row, geb, preferred_element_type=f32)
        return jnp.sum(c, axis=1, keepdims=True)

    xb = [x_ref[g] for g in rng]
    m = [jnp.mean(xb[g], axis=-1, keepdims=True) for g in rng]
    var = [jnp.mean(xb[g] * xb[g], axis=-1, keepdims=True) - m[g] * m[g]
           for g in rng]
    nx = [(xb[g] - m[g]) * lax.rsqrt(var[g] + 1e-5) * lng_ref[...]
          + lnb_ref[...] for g in rng]
    lane = lax.broadcasted_iota(jnp.int32, (1, DIM), 1)
    nx2b = [jnp.where(lane < DIM // 2,
                      jnp.concatenate([jnp.zeros((1, DIM), f32), nx[g][:-1]],
                                      axis=0),
                      nx[g]).astype(bf16) for g in rng]
    v = [(jnp.dot(nx2b[g], Whv_ref[...], preferred_element_type=f32)
          + bhv_ref[...]).astype(bf16) for g in rng]
    v = [v[g] * (1.0 / (1.0 + jnp.exp(-v[g]))) for g in rng]
    gate = [(jnp.dot(nx2b[g], Whg_ref[...], preferred_element_type=f32)
             + bhg_ref[...]).astype(bf16) for g in rng]
    gate = [gate[g] * (1.0 / (1.0 + jnp.exp(-gate[g]))) for g in rng]
    absg = [jnp.abs(gate[g]) for g in rng]
    gmax = [jnp.max(jnp.max(gate[g], axis=0, keepdims=True), axis=1,
                    keepdims=True) for g in rng]
    floor_g = [jnp.floor(gmax[g]).astype(f32) for g in rng]

    qk = [(jnp.dot(nx2b[g], Wqk_ref[...], preferred_element_type=f32)
           + bqk_ref[...]).astype(bf16) for g in rng]
    qk = [qk[g] * (1.0 / (1.0 + jnp.exp(-qk[g]))) for g in rng]
    q0 = [qk[g] * gamma_ref[0:1, :] + beta_ref[0:1, :] for g in rng]
    k0 = [qk[g] * gamma_ref[1:2, :] + beta_ref[1:2, :] for g in rng]
    cs = cs_ref[...]
    sn = sn_ref[...]
    q = [q0[g] * cs + jnp.dot(q0[g], R_ref[...],
                              preferred_element_type=f32).astype(bf16) * sn
         for g in rng]
    k = [k0[g] * cs + jnp.dot(k0[g], R_ref[...],
                              preferred_element_type=f32).astype(bf16) * sn
         for g in rng]

    cnt1 = [_count_ge(absg[g], one_b) for g in rng]

    def _search_all():
        outs = []
        for g in rng:
            lo = jnp.zeros((1, 1), f32)
            hi = jnp.full((1, 1), 128.0, f32)
            for _ in range(8):
                mid = jnp.floor((lo + hi + 1.0) * 0.5)
                cnt = _count_ge(absg[g], mid.astype(bf16))
                ok = cnt > 90000.0
                lo = jnp.where(ok, mid, lo)
                hi = jnp.where(ok, hi, mid - 1.0)
            outs.append(lo)
        return tuple(outs)

    need = cnt1[0][0, 0]
    for g in range(1, G):
        need = jnp.maximum(need, cnt1[g][0, 0])
    tstar = lax.cond(need > 90000.0, _search_all,
                     lambda: tuple(jnp.zeros((1, 1), f32) for _ in rng))

    trim = [jnp.maximum(jnp.minimum(tstar[g], floor_g[g]), 1.0) for g in rng]
    ind = [jnp.where(absg[g] >= trim[g].astype(bf16), one_b, zero_b)
           for g in rng]
    counts = [jnp.dot(A4_ref[...],
                      jnp.dot(ind[g], B4_ref[...],
                              preferred_element_type=f32).astype(bf16),
                      preferred_element_type=f32) for g in rng]
    cmax = [jnp.max(jnp.max(counts[g], axis=0, keepdims=True), axis=1,
                    keepdims=True) for g in rng]
    t2star = [jnp.zeros((1, 1), f32) for g in rng]
    for t in range(1, 17):
        ct = [jnp.sum(jnp.sum(jnp.where(counts[g] >= float(t), 1.0, 0.0),
                              axis=0, keepdims=True), axis=1, keepdims=True)
              for g in rng]
        t2star = [t2star[g] + jnp.where(ct[g] > 3600.0, 1.0, 0.0) for g in rng]
    t2 = [jnp.where(t2star[g] >= 0.5, t2star[g], cmax[g]) for g in rng]
    bv = [jnp.where(counts[g] >= t2[g], 1.0, 0.25).astype(bf16) for g in rng]
    gm = [jnp.dot(A4T_ref[...],
                  jnp.dot(bv[g], B4T_ref[...],
                          preferred_element_type=f32).astype(bf16),
                  preferred_element_type=f32).astype(bf16) for g in rng]

    sim = [lax.dot_general(q[g], k[g], (((1,), (1,)), ((), ())),
                           preferred_element_type=f32) + bias_ref[...]
           for g in rng]
    a = [jnp.maximum(sim[g].astype(bf16), jnp.bfloat16(0.0)) for g in rng]
    attn = [a[g] * a[g] * mask2_ref[...] for g in rng]
    out = [jnp.dot(attn[g], v[g], preferred_element_type=f32).astype(bf16)
           for g in rng]
    out = [gm[g] * out[g] * gate[g] for g in rng]
    for g in rng:
        o_ref[g] = (jnp.dot(out[g], Wout_ref[...], preferred_element_type=f32)
                    + bout_ref[...] + xb[g])


def kernel(x, my_mask2, ln_g, ln_b, Wh, bh, Wqk, bqk, gamma, beta, rel_emb, Wout, bout):
    f32 = jnp.float32

    inv = 1.0 / (10000.0 ** (jnp.arange(0, ROT, 2, dtype=f32) / ROT))
    fr = jnp.repeat(jnp.arange(S, dtype=f32)[:, None] * inv[None, :], 2, axis=-1)
    cs = jnp.concatenate([jnp.cos(fr), jnp.ones((S, QK - ROT), f32)], axis=1)
    sn = jnp.concatenate([jnp.sin(fr), jnp.zeros((S, QK - ROT), f32)], axis=1)
    Rnp = np.zeros((QK, QK), np.float32)
    for i in range(0, ROT, 2):
        Rnp[i + 1, i] = -1.0
        Rnp[i, i + 1] = 1.0
    R = jnp.asarray(Rnp, jnp.bfloat16)

    A4np = np.zeros((100, S), np.float32)
    for r in range(4):
        A4np[np.arange(100), 5 * np.arange(100) + r] = 1.0
    B4np = np.zeros((HID, 120), np.float32)
    for c in range(4):
        B4np[5 * np.arange(120) + c, np.arange(120)] = 1.0
    A4 = jnp.asarray(A4np, jnp.bfloat16)
    B4 = jnp.asarray(B4np, jnp.bfloat16)
    A4T = jnp.asarray(np.ascontiguousarray(A4np.T), jnp.bfloat16)
    B4T = jnp.asarray(np.ascontiguousarray(B4np.T), jnp.bfloat16)

    nb = NB // 2
    pos = jnp.arange(S)
    n = pos[:, None] - pos[None, :]
    ret = (n < 0).astype(jnp.int32) * nb
    na = jnp.abs(n)
    max_exact = nb // 2
    vil = max_exact + (jnp.log(jnp.maximum(na, 1).astype(f32) / max_exact)
                       / np.float32(np.log(128.0 / max_exact))
                       * (nb - max_exact)).astype(jnp.int32)
    vil = jnp.minimum(vil, nb - 1)
    bucket = (ret + jnp.where(na < max_exact, na, vil)).astype(jnp.int32)

    emb_row = (rel_emb[:, 0] * np.float32(QK ** 0.5)).reshape(1, NB)
    bias = pl.pallas_call(
        _bias_kernel,
        out_shape=jax.ShapeDtypeStruct((S, S), f32),
        name="t5_bias",
    )(emb_row, bucket)

    bf16 = jnp.bfloat16
    Whv, Whg = Wh[:, :HID].astype(bf16), Wh[:, HID:].astype(bf16)
    bhv, bhg = bh[:HID].reshape(1, HID), bh[HID:].reshape(1, HID)
    lng, lnb = ln_g.reshape(1, DIM), ln_b.reshape(1, DIM)
    bqk2 = bqk.reshape(1, QK)
    bout2 = bout.reshape(1, DIM)

    def full(a):
        nd = a.ndim
        return pl.BlockSpec(a.shape, lambda b, _n=nd: (0,) * _n)

    mask2s = (my_mask2 * np.float32(1.0 / (S * S))).astype(bf16)
    consts = (mask2s, bias, lng, lnb, Whv, bhv, Whg, bhg, Wqk.astype(bf16), bqk2,
              gamma.astype(bf16), beta.astype(bf16), cs.astype(bf16), sn.astype(bf16),
              R, A4, B4, A4T, B4T, Wout.astype(jnp.bfloat16), bout2)
    in_specs = [pl.BlockSpec((G, S, DIM), lambda b: (b, 0, 0))]
    in_specs += [full(a) for a in consts]

    out = pl.pallas_call(
        _gau_kernel,
        grid=(B // G,),
        in_specs=in_specs,
        out_specs=pl.BlockSpec((G, S, DIM), lambda b: (b, 0, 0)),
        out_shape=jax.ShapeDtypeStruct((B, S, DIM), f32),
        compiler_params=pltpu.CompilerParams(
            dimension_semantics=("arbitrary",),
            vmem_limit_bytes=48 * 1024 * 1024,
        ),
        name="gau_fused",
    )(x, *consts)
    return out


# speedup vs baseline: 46.2801x; 1.0591x over previous
import numpy as np
import jax
import jax.numpy as jnp
from jax import lax
from jax.experimental import pallas as pl
from jax.experimental.pallas import tpu as pltpu

B, S, DIM = 20, 500, 300
HID = 600
QK = 128
ROT = 32
NB = 32


def _prep_kernel(emb_ref, bucket_ref, mask2_ref, Wh_ref, Wqk_ref, Wout_ref,
                 gamma_ref, beta_ref, bh_ref, bqk_ref,
                 bias_o, whv_o, whg_o, wqk_o, wout_o, m2_o,
                 gb_o, bb_o, bhv_o, bhg_o, bqk_o):
    bf16 = jnp.bfloat16
    emb = jnp.broadcast_to(emb_ref[...], (S, NB))
    bias_o[...] = jnp.take_along_axis(emb, bucket_ref[...], axis=1)
    whv_o[...] = Wh_ref[:, :HID].astype(bf16)
    whg_o[...] = Wh_ref[:, HID:].astype(bf16)
    wqk_o[...] = Wqk_ref[...].astype(bf16)
    wout_o[...] = Wout_ref[...].astype(bf16)
    m2_o[...] = (mask2_ref[...] * np.float32(1.0 / (S * S))).astype(bf16)
    gb_o[...] = gamma_ref[...].astype(bf16)
    bb_o[...] = beta_ref[...].astype(bf16)
    bhv_o[...] = bh_ref[:, :HID].astype(bf16)
    bhg_o[...] = bh_ref[:, HID:].astype(bf16)
    bqk_o[...] = bqk_ref[...].astype(bf16)


G = 2


def _gau_kernel(x_ref, mask2_ref, bias_ref, lng_ref, lnb_ref,
                Whv_ref, bhv_ref, Whg_ref, bhg_ref,
                Wqk_ref, bqk_ref, gamma_ref, beta_ref,
                cs_ref, sn_ref, R_ref,
                A4_ref, B4_ref, A4T_ref, B4T_ref,
                Wout_ref, bout_ref, o_ref):
    f32 = jnp.float32
    bf16 = jnp.bfloat16
    rng = range(G)
    one_b = jnp.bfloat16(1.0)
    zero_b = jnp.bfloat16(0.0)
    ones_row = jnp.ones((1, S), bf16)

    def _silu(x):
        y = x * 0.5
        return y + y * jnp.tanh(y)

    def _count_ge(absg, thresh_bf):
        geb = jnp.where(absg >= thresh_bf, one_b, zero_b)
        c = jnp.dot(ones_row, geb, preferred_element_type=f32)
        return jnp.sum(c, axis=1, keepdims=True)

    xb = [x_ref[g] for g in rng]
    m = [jnp.mean(xb[g], axis=-1, keepdims=True) for g in rng]
    var = [jnp.mean(xb[g] * xb[g], axis=-1, keepdims=True) - m[g] * m[g]
           for g in rng]
    nx = [(xb[g] - m[g]) * lax.rsqrt(var[g] + 1e-5) * lng_ref[...]
          + lnb_ref[...] for g in rng]
    lane = lax.broadcasted_iota(jnp.int32, (1, DIM), 1)
    nx2b = [jnp.where(lane < DIM // 2,
                      jnp.concatenate([jnp.zeros((1, DIM), f32), nx[g][:-1]],
                                      axis=0),
                      nx[g]).astype(bf16) for g in rng]
    v = [jnp.dot(nx2b[g], Whv_ref[...],
                 preferred_element_type=f32).astype(bf16) + bhv_ref[...]
         for g in rng]
    v = [_silu(v[g]) for g in rng]
    gate = [jnp.dot(nx2b[g], Whg_ref[...],
                    preferred_element_type=f32).astype(bf16) + bhg_ref[...]
            for g in rng]
    gate = [_silu(gate[g]) for g in rng]
    absg = [jnp.abs(gate[g]) for g in rng]
    gmax = [jnp.max(jnp.max(gate[g], axis=0, keepdims=True), axis=1,
                    keepdims=True) for g in rng]
    floor_g = [jnp.floor(gmax[g]).astype(f32) for g in rng]

    qk = [jnp.dot(nx2b[g], Wqk_ref[...],
                  preferred_element_type=f32).astype(bf16) + bqk_ref[...]
          for g in rng]
    qk = [_silu(qk[g]) for g in rng]
    q0 = [qk[g] * gamma_ref[0:1, :] + beta_ref[0:1, :] for g in rng]
    k0 = [qk[g] * gamma_ref[1:2, :] + beta_ref[1:2, :] for g in rng]
    cs = cs_ref[...]
    sn = sn_ref[...]
    q = [q0[g] * cs + jnp.dot(q0[g], R_ref[...],
                              preferred_element_type=f32).astype(bf16) * sn
         for g in rng]
    k = [k0[g] * cs + jnp.dot(k0[g], R_ref[...],
                              preferred_element_type=f32).astype(bf16) * sn
         for g in rng]

    cnt1 = [_count_ge(absg[g], one_b) for g in rng]

    def _search_all():
        outs = []
        for g in rng:
            lo = jnp.zeros((1, 1), f32)
            hi = jnp.full((1, 1), 128.0, f32)
            for _ in range(8):
                mid = jnp.floor((lo + hi + 1.0) * 0.5)
                cnt = _count_ge(absg[g], mid.astype(bf16))
                ok = cnt > 90000.0
                lo = jnp.where(ok, mid, lo)
                hi = jnp.where(ok, hi, mid - 1.0)
            outs.append(lo)
        return tuple(outs)

    need = cnt1[0][0, 0]
    for g in range(1, G):
        need = jnp.maximum(need, cnt1[g][0, 0])
    tstar = lax.cond(need > 90000.0, _search_all,
                     lambda: tuple(jnp.zeros((1, 1), f32) for _ in rng))

    trim = [jnp.maximum(jnp.minimum(tstar[g], floor_g[g]), 1.0) for g in rng]
    ind = [jnp.where(absg[g] >= trim[g].astype(bf16), one_b, zero_b)
           for g in rng]
    counts = [jnp.dot(A4_ref[...],
                      jnp.dot(ind[g], B4_ref[...],
                              preferred_element_type=f32).astype(bf16),
                      preferred_element_type=f32) for g in rng]
    cmax = [jnp.max(jnp.max(counts[g], axis=0, keepdims=True), axis=1,
                    keepdims=True) for g in rng]
    t2star = [jnp.zeros((1, 1), f32) for g in rng]
    for t in range(1, 17):
        ct = [jnp.sum(jnp.sum(jnp.where(counts[g] >= float(t), 1.0, 0.0),
                              axis=0, keepdims=True), axis=1, keepdims=True)
              for g in rng]
        t2star = [t2star[g] + jnp.where(ct[g] > 3600.0, 1.0, 0.0) for g in rng]
    t2 = [jnp.where(t2star[g] >= 0.5, t2star[g], cmax[g]) for g in rng]
    bv = [jnp.where(counts[g] >= t2[g], 1.0, 0.25).astype(bf16) for g in rng]
    gm = [jnp.dot(A4T_ref[...],
                  jnp.dot(bv[g], B4T_ref[...],
                          preferred_element_type=f32).astype(bf16),
                  preferred_element_type=f32).astype(bf16) for g in rng]

    sim = [lax.dot_general(q[g], k[g], (((1,), (1,)), ((), ())),
                           preferred_element_type=f32) + bias_ref[...]
           for g in rng]
    a = [jnp.maximum(sim[g].astype(bf16), jnp.bfloat16(0.0)) for g in rng]
    attn = [a[g] * a[g] * mask2_ref[...] for g in rng]
    out = [jnp.dot(attn[g], v[g], preferred_element_type=f32).astype(bf16)
           for g in rng]
    out = [gm[g] * out[g] * gate[g] for g in rng]
    for g in rng:
        o_ref[g] = (jnp.dot(out[g], Wout_ref[...], preferred_element_type=f32)
                    + bout_ref[...] + xb[g])


def kernel(x, my_mask2, ln_g, ln_b, Wh, bh, Wqk, bqk, gamma, beta, rel_emb, Wout, bout):
    f32 = jnp.float32

    inv = 1.0 / (10000.0 ** (jnp.arange(0, ROT, 2, dtype=f32) / ROT))
    fr = jnp.repeat(jnp.arange(S, dtype=f32)[:, None] * inv[None, :], 2, axis=-1)
    cs = jnp.concatenate([jnp.cos(fr), jnp.ones((S, QK - ROT), f32)], axis=1)
    sn = jnp.concatenate([jnp.sin(fr), jnp.zeros((S, QK - ROT), f32)], axis=1)
    Rnp = np.zeros((QK, QK), np.float32)
    for i in range(0, ROT, 2):
        Rnp[i + 1, i] = -1.0
        Rnp[i, i + 1] = 1.0
    R = jnp.asarray(Rnp, jnp.bfloat16)

    A4np = np.zeros((100, S), np.float32)
    for r in range(4):
        A4np[np.arange(100), 5 * np.arange(100) + r] = 1.0
    B4np = np.zeros((HID, 120), np.float32)
    for c in range(4):
        B4np[5 * np.arange(120) + c, np.arange(120)] = 1.0
    A4 = jnp.asarray(A4np, jnp.bfloat16)
    B4 = jnp.asarray(B4np, jnp.bfloat16)
    A4T = jnp.asarray(np.ascontiguousarray(A4np.T), jnp.bfloat16)
    B4T = jnp.asarray(np.ascontiguousarray(B4np.T), jnp.bfloat16)

    nb = NB // 2
    pos = jnp.arange(S)
    n = pos[:, None] - pos[None, :]
    ret = (n < 0).astype(jnp.int32) * nb
    na = jnp.abs(n)
    max_exact = nb // 2
    vil = max_exact + (jnp.log(jnp.maximum(na, 1).astype(f32) / max_exact)
                       / np.float32(np.log(128.0 / max_exact))
                       * (nb - max_exact)).astype(jnp.int32)
    vil = jnp.minimum(vil, nb - 1)
    bucket = (ret + jnp.where(na < max_exact, na, vil)).astype(jnp.int32)

    bf16 = jnp.bfloat16
    emb_row = (rel_emb[:, 0] * np.float32(QK ** 0.5)).reshape(1, NB)
    sds = jax.ShapeDtypeStruct
    bias, Whv, Whg, Wqkb, Woutb, mask2s, gammab, betab, bhv, bhg, bqk2 = (
        pl.pallas_call(
            _prep_kernel,
            out_shape=(sds((S, S), f32), sds((DIM, HID), bf16),
                       sds((DIM, HID), bf16), sds((DIM, QK), bf16),
                       sds((HID, DIM), bf16), sds((S, S), bf16),
                       sds((2, QK), bf16), sds((2, QK), bf16),
                       sds((1, HID), bf16), sds((1, HID), bf16),
                       sds((1, QK), bf16)),
            name="gau_prep",
        )(emb_row, bucket, my_mask2, Wh, Wqk, Wout, gamma, beta,
          bh.reshape(1, 2 * HID), bqk.reshape(1, QK)))

    lng, lnb = ln_g.reshape(1, DIM), ln_b.reshape(1, DIM)
    bout2 = bout.reshape(1, DIM)

    def full(a):
        nd = a.ndim
        return pl.BlockSpec(a.shape, lambda b, _n=nd: (0,) * _n)

    consts = (mask2s, bias, lng, lnb, Whv, bhv, Whg, bhg, Wqkb, bqk2,
              gammab, betab, cs.astype(bf16), sn.astype(bf16),
              R, A4, B4, A4T, B4T, Woutb, bout2)
    in_specs = [pl.BlockSpec((G, S, DIM), lambda b: (b, 0, 0))]
    in_specs += [full(a) for a in consts]

    out = pl.pallas_call(
        _gau_kernel,
        grid=(B // G,),
        in_specs=in_specs,
        out_specs=pl.BlockSpec((G, S, DIM), lambda b: (b, 0, 0)),
        out_shape=jax.ShapeDtypeStruct((B, S, DIM), f32),
        compiler_params=pltpu.CompilerParams(
            dimension_semantics=("arbitrary",),
            vmem_limit_bytes=48 * 1024 * 1024,
        ),
        name="gau_fused",
    )(x, *consts)
    return out
